# baseline (speedup 1.0000x reference)
import functools

import jax
import jax.numpy as jnp
from jax.experimental import pallas as pl

_F32 = jnp.float32
_BF16 = jnp.bfloat16
_COST_LAMBDA = 0.0005


def _gelu(v):
    return 0.5 * v * (1.0 + jax.lax.erf(v * (2.0 ** -0.5)))


def _dot_t(a, b):
    return jax.lax.dot_general(a.astype(_BF16), b.astype(_BF16),
                               (((1,), (1,)), ((), ())),
                               preferred_element_type=_F32)


def _dot(a, b):
    return jax.lax.dot_general(a.astype(_BF16), b.astype(_BF16),
                               (((1,), (0,)), ((), ())),
                               preferred_element_type=_F32)


def _k1_body(x_ref, rmsw_ref, wqkv_ref, wd_ref, bd_ref, wr_ref, breff_ref,
             h_ref, q_ref, k_ref, v_ref, g_ref, w_ref):
    xb = x_ref[...]
    d = xb.shape[-1]
    norm = jnp.sqrt(jnp.sum(xb * xb, axis=-1, keepdims=True)) * (d ** -0.5)
    hb = rmsw_ref[...] * xb / (norm + 1e-8)
    h_ref[...] = hb

    qkv = _dot_t(hb, wqkv_ref[...])
    q_ref[...] = qkv[:, :d].astype(_BF16)
    k_ref[...] = qkv[:, d:2 * d].astype(_BF16)
    v_ref[...] = qkv[:, 2 * d:].astype(_BF16)

    hd = _dot_t(hb, wd_ref[...]) + bd_ref[...]
    g_ref[...] = _gelu(hd).astype(_BF16)

    logits = _dot_t(hb, wr_ref[...]) + breff_ref[...]
    e = logits.shape[-1]
    iota = jax.lax.broadcasted_iota(jnp.int32, logits.shape, 1)
    l1 = jnp.max(logits, axis=-1, keepdims=True)
    a1 = jnp.min(jnp.where(logits == l1, iota, e), axis=-1, keepdims=True)
    masked = jnp.where(iota == a1, -jnp.inf, logits)
    l2 = jnp.max(masked, axis=-1, keepdims=True)
    a2 = jnp.min(jnp.where(masked == l2, iota, e), axis=-1, keepdims=True)
    z = jnp.sum(jnp.exp(logits - l1), axis=-1, keepdims=True)
    p1 = 1.0 / z
    p2 = jnp.exp(l2 - l1) / z
    e2 = jnp.exp(p2 - p1)
    inv = 1.0 / (1.0 + e2)
    w_ref[...] = jnp.where(iota == a1, inv, 0.0) + jnp.where(iota == a2, e2 * inv, 0.0)


def _k2_body(q_ref, k_ref, v_ref, x_ref, wo_ref, o_ref, *, scale):
    qb = q_ref[0]
    scores = _dot_t(qb, k_ref[0]) * scale
    m = jnp.max(scores, axis=-1, keepdims=True)
    p = jnp.exp(scores - m)
    att = p / jnp.sum(p, axis=-1, keepdims=True)
    o = _dot(att, v_ref[0])
    o_ref[0] = x_ref[0] + _dot_t(o, wo_ref[...])


def _k3_body(*refs, n_experts):
    (y1_ref, h_ref, g_ref, w_ref) = refs[:4]
    eref = refs[4:4 + 4 * n_experts]
    (ws1_ref, bs1_ref, ws2_ref, bs2_ref, wu_ref, bu_ref, wc_ref, bc_ref,
     out_ref) = refs[4 + 4 * n_experts:]

    gb = g_ref[...]
    wb = w_ref[...]
    acc = jnp.zeros((gb.shape[0], gb.shape[1]), _F32)
    ctot = jnp.zeros((1, gb.shape[1]), _F32)
    for i in range(n_experts):
        w1 = eref[4 * i][...]
        b1 = eref[4 * i + 1][...]
        w2 = eref[4 * i + 2][...]
        b2 = eref[4 * i + 3][...]
        gb1 = _gelu(b1)
        t = _gelu(_dot_t(gb, w1) + b1) - gb1
        acc = acc + _dot_t(wb[:, i:i + 1] * t, w2)
        ctot = ctot + _dot_t(gb1, w2) + b2

    s = _gelu(_dot_t(gb, ws1_ref[...]) + bs1_ref[...])
    s = _dot_t(s, ws2_ref[...]) + bs2_ref[...]
    moe = acc + ctot + 0.1 * s

    up = _dot_t(moe, wu_ref[...]) + bu_ref[...]
    core = _dot_t(_gelu(h_ref[...]), wc_ref[...]) + bc_ref[...]
    out_ref[...] = y1_ref[...] + up + core


def kernel(x, rms_w, Wqkv, Wo, Wd, bd, Wu, bu, Wr, br, expert_params,
           Ws1, bs1, Ws2, bs2, Wc, bc):
    B, T, D = x.shape
    N = B * T
    L = Wd.shape[0]
    E = Wr.shape[0]
    hdims = [int(w1.shape[0]) for (w1, _, _, _) in expert_params]
    cost = jnp.asarray([2 * L * hd for hd in hdims], _F32)
    br_eff = (br - _COST_LAMBDA * cost).reshape(1, E)

    x2 = x.reshape(N, D)
    r2 = lambda a: a.reshape(1, -1)
    bf = lambda a: a.astype(_BF16)

    R1 = 256
    full = lambda arr: pl.BlockSpec(arr.shape, lambda i: (0,) * arr.ndim)
    row = lambda c: pl.BlockSpec((R1, c), lambda i: (i, 0))

    h, q, k, v, g, w = pl.pallas_call(
        _k1_body,
        grid=(N // R1,),
        in_specs=[row(D), full(r2(rms_w)), full(Wqkv), full(Wd),
                  full(r2(bd)), full(Wr), full(br_eff)],
        out_specs=[row(D), row(D), row(D), row(D), row(L), row(E)],
        out_shape=[
            jax.ShapeDtypeStruct((N, D), _F32),
            jax.ShapeDtypeStruct((N, D), _BF16),
            jax.ShapeDtypeStruct((N, D), _BF16),
            jax.ShapeDtypeStruct((N, D), _BF16),
            jax.ShapeDtypeStruct((N, L), _BF16),
            jax.ShapeDtypeStruct((N, E), _F32),
        ],
    )(x2, r2(rms_w), bf(Wqkv), bf(Wd), r2(bd), bf(Wr), br_eff)

    RQ = 256
    q3 = q.reshape(B, T, D)
    k3 = k.reshape(B, T, D)
    v3 = v.reshape(B, T, D)
    qblk = pl.BlockSpec((1, RQ, D), lambda b, i: (b, i, 0))
    kvblk = pl.BlockSpec((1, T, D), lambda b, i: (b, 0, 0))
    y1 = pl.pallas_call(
        functools.partial(_k2_body, scale=D ** -0.5),
        grid=(B, T // RQ),
        in_specs=[qblk, kvblk, kvblk, qblk,
                  pl.BlockSpec(Wo.shape, lambda b, i: (0, 0))],
        out_specs=qblk,
        out_shape=jax.ShapeDtypeStruct((B, T, D), _F32),
    )(q3, k3, v3, x, bf(Wo))

    R3 = 256
    row3 = lambda c: pl.BlockSpec((R3, c), lambda i: (i, 0))
    eops, especs = [], []
    for (w1, b1, w2, b2) in expert_params:
        for a in (bf(w1), r2(b1), bf(w2), r2(b2)):
            eops.append(a)
            especs.append(full(a))
    out = pl.pallas_call(
        functools.partial(_k3_body, n_experts=E),
        grid=(N // R3,),
        in_specs=[row3(D), row3(D), row3(L), row3(E)] + especs + [
            full(Ws1), full(r2(bs1)), full(Ws2), full(r2(bs2)),
            full(Wu), full(r2(bu)), full(Wc), full(r2(bc))],
        out_specs=row3(D),
        out_shape=jax.ShapeDtypeStruct((N, D), _F32),
    )(y1.reshape(N, D), h, g, w, *eops,
      bf(Ws1), r2(bs1), bf(Ws2), r2(bs2), bf(Wu), r2(bu), bf(Wc), r2(bc))

    return out.reshape(B, T, D)


# speedup vs baseline: 1.3166x; 1.3166x over previous
import functools

import jax
import jax.numpy as jnp
from jax.experimental import pallas as pl

_F32 = jnp.float32
_BF16 = jnp.bfloat16
_COST_LAMBDA = 0.0005


def _gelu(v):
    return 0.5 * v * (1.0 + jax.lax.erf(v * (2.0 ** -0.5)))


def _dot_t(a, b):
    return jax.lax.dot_general(a, b, (((1,), (1,)), ((), ())),
                               preferred_element_type=_F32)


def _dot(a, b):
    return jax.lax.dot_general(a, b, (((1,), (0,)), ((), ())),
                               preferred_element_type=_F32)


def _k1_body(x_ref, rmsw_ref, wqkv_ref, wd_ref, bd_ref, wr_ref, breff_ref,
             h_ref, q_ref, k_ref, v_ref, g_ref, w_ref):
    xb = x_ref[...]
    d = xb.shape[-1]
    norm = jnp.sqrt(jnp.sum(xb * xb, axis=-1, keepdims=True)) * (d ** -0.5)
    hb = rmsw_ref[...] * xb / (norm + 1e-8)
    h_ref[...] = hb

    qkv = _dot_t(hb, wqkv_ref[...])
    q_ref[...] = qkv[:, :d]
    k_ref[...] = qkv[:, d:2 * d]
    v_ref[...] = qkv[:, 2 * d:]

    hd = _dot_t(hb, wd_ref[...]) + bd_ref[...]
    g_ref[...] = _gelu(hd)

    logits = _dot_t(hb, wr_ref[...]) + breff_ref[...]
    e = logits.shape[-1]
    iota = jax.lax.broadcasted_iota(jnp.int32, logits.shape, 1)
    l1 = jnp.max(logits, axis=-1, keepdims=True)
    a1 = jnp.min(jnp.where(logits == l1, iota, e), axis=-1, keepdims=True)
    masked = jnp.where(iota == a1, -jnp.inf, logits)
    l2 = jnp.max(masked, axis=-1, keepdims=True)
    a2 = jnp.min(jnp.where(masked == l2, iota, e), axis=-1, keepdims=True)
    z = jnp.sum(jnp.exp(logits - l1), axis=-1, keepdims=True)
    p1 = 1.0 / z
    p2 = jnp.exp(l2 - l1) / z
    e2 = jnp.exp(p2 - p1)
    inv = 1.0 / (1.0 + e2)
    w_ref[...] = jnp.where(iota == a1, inv, 0.0) + jnp.where(iota == a2, e2 * inv, 0.0)


def _k2_body(q_ref, k_ref, v_ref, x_ref, wo_ref, o_ref, *, scale):
    qb = q_ref[0]
    scores = _dot_t(qb, k_ref[0]) * scale
    m = jnp.max(scores, axis=-1, keepdims=True)
    p = jnp.exp(scores - m)
    att = p / jnp.sum(p, axis=-1, keepdims=True)
    o = _dot(att, v_ref[0])
    o_ref[0] = x_ref[0] + _dot_t(o, wo_ref[...])


def _k3_body(*refs, n_experts):
    (y1_ref, h_ref, g_ref, w_ref) = refs[:4]
    eref = refs[4:4 + 4 * n_experts]
    (ws1_ref, bs1_ref, ws2_ref, bs2_ref, wu_ref, bu_ref, wc_ref, bc_ref,
     out_ref) = refs[4 + 4 * n_experts:]

    gb = g_ref[...]
    wb = w_ref[...]
    acc = jnp.zeros((gb.shape[0], gb.shape[1]), _F32)
    ctot = jnp.zeros((1, gb.shape[1]), _F32)
    for i in range(n_experts):
        w1 = eref[4 * i][...]
        b1 = eref[4 * i + 1][...]
        w2 = eref[4 * i + 2][...]
        b2 = eref[4 * i + 3][...]
        gb1 = _gelu(b1)
        t = _gelu(_dot_t(gb, w1) + b1) - gb1
        acc = acc + _dot_t(wb[:, i:i + 1] * t, w2)
        ctot = ctot + _dot_t(gb1, w2) + b2

    s = _gelu(_dot_t(gb, ws1_ref[...]) + bs1_ref[...])
    s = _dot_t(s, ws2_ref[...]) + bs2_ref[...]
    moe = acc + ctot + 0.1 * s

    up = _dot_t(moe, wu_ref[...]) + bu_ref[...]
    core = _dot_t(_gelu(h_ref[...]), wc_ref[...]) + bc_ref[...]
    out_ref[...] = y1_ref[...] + up + core


def kernel(x, rms_w, Wqkv, Wo, Wd, bd, Wu, bu, Wr, br, expert_params,
           Ws1, bs1, Ws2, bs2, Wc, bc):
    B, T, D = x.shape
    N = B * T
    L = Wd.shape[0]
    E = Wr.shape[0]
    hdims = [int(w1.shape[0]) for (w1, _, _, _) in expert_params]
    cost = jnp.asarray([2 * L * hd for hd in hdims], _F32)
    br_eff = (br - _COST_LAMBDA * cost).reshape(1, E)

    x2 = x.reshape(N, D)
    r2 = lambda a: a.reshape(1, -1)

    R1 = 512
    full = lambda arr: pl.BlockSpec(arr.shape, lambda i: (0,) * arr.ndim)
    row = lambda c: pl.BlockSpec((R1, c), lambda i: (i, 0))

    h, q, k, v, g, w = pl.pallas_call(
        _k1_body,
        grid=(N // R1,),
        in_specs=[row(D), full(r2(rms_w)), full(Wqkv), full(Wd),
                  full(r2(bd)), full(Wr), full(br_eff)],
        out_specs=[row(D), row(D), row(D), row(D), row(L), row(E)],
        out_shape=[
            jax.ShapeDtypeStruct((N, D), _F32),
            jax.ShapeDtypeStruct((N, D), _F32),
            jax.ShapeDtypeStruct((N, D), _F32),
            jax.ShapeDtypeStruct((N, D), _F32),
            jax.ShapeDtypeStruct((N, L), _F32),
            jax.ShapeDtypeStruct((N, E), _F32),
        ],
    )(x2, r2(rms_w), Wqkv, Wd, r2(bd), Wr, br_eff)

    RQ = 512
    q3 = q.reshape(B, T, D)
    k3 = k.reshape(B, T, D)
    v3 = v.reshape(B, T, D)
    qblk = pl.BlockSpec((1, RQ, D), lambda b, i: (b, i, 0))
    kvblk = pl.BlockSpec((1, T, D), lambda b, i: (b, 0, 0))
    y1 = pl.pallas_call(
        functools.partial(_k2_body, scale=D ** -0.5),
        grid=(B, T // RQ),
        in_specs=[qblk, kvblk, kvblk, qblk,
                  pl.BlockSpec(Wo.shape, lambda b, i: (0, 0))],
        out_specs=qblk,
        out_shape=jax.ShapeDtypeStruct((B, T, D), _F32),
    )(q3, k3, v3, x, Wo)

    R3 = 512
    row3 = lambda c: pl.BlockSpec((R3, c), lambda i: (i, 0))
    eops, especs = [], []
    for (w1, b1, w2, b2) in expert_params:
        for a in (w1, r2(b1), w2, r2(b2)):
            eops.append(a)
            especs.append(full(a))
    out = pl.pallas_call(
        functools.partial(_k3_body, n_experts=E),
        grid=(N // R3,),
        in_specs=[row3(D), row3(D), row3(L), row3(E)] + especs + [
            full(Ws1), full(r2(bs1)), full(Ws2), full(r2(bs2)),
            full(Wu), full(r2(bu)), full(Wc), full(r2(bc))],
        out_specs=row3(D),
        out_shape=jax.ShapeDtypeStruct((N, D), _F32),
    )(y1.reshape(N, D), h, g, w, *eops,
      Ws1, r2(bs1), Ws2, r2(bs2), Wu, r2(bu), Wc, r2(bc))

    return out.reshape(B, T, D)


# speedup vs baseline: 1.3446x; 1.0213x over previous
import functools

import jax
import jax.numpy as jnp
from jax.experimental import pallas as pl

_F32 = jnp.float32
_BF16 = jnp.bfloat16
_COST_LAMBDA = 0.0005


def _gelu(v):
    return 0.5 * v * (1.0 + jax.lax.erf(v * (2.0 ** -0.5)))


def _dot_t(a, b):
    return jax.lax.dot_general(a, b, (((1,), (1,)), ((), ())),
                               preferred_element_type=_F32)


def _dot(a, b):
    return jax.lax.dot_general(a, b, (((1,), (0,)), ((), ())),
                               preferred_element_type=_F32)


def _k1_body(x_ref, rmsw_ref, wqkv_ref, wd_ref, bd_ref, wr_ref, breff_ref,
             h_ref, q_ref, k_ref, v_ref, g_ref, w_ref):
    xb = x_ref[...]
    d = xb.shape[-1]
    norm = jnp.sqrt(jnp.sum(xb * xb, axis=-1, keepdims=True)) * (d ** -0.5)
    hb = rmsw_ref[...] * xb / (norm + 1e-8)
    h_ref[...] = hb

    qkv = _dot_t(hb, wqkv_ref[...])
    q_ref[...] = qkv[:, :d].astype(_BF16)
    k_ref[...] = qkv[:, d:2 * d].astype(_BF16)
    v_ref[...] = qkv[:, 2 * d:].astype(_BF16)

    hd = _dot_t(hb, wd_ref[...]) + bd_ref[...]
    g_ref[...] = _gelu(hd).astype(_BF16)

    logits = _dot_t(hb, wr_ref[...]) + breff_ref[...]
    e = logits.shape[-1]
    iota = jax.lax.broadcasted_iota(jnp.int32, logits.shape, 1)
    l1 = jnp.max(logits, axis=-1, keepdims=True)
    a1 = jnp.min(jnp.where(logits == l1, iota, e), axis=-1, keepdims=True)
    masked = jnp.where(iota == a1, -jnp.inf, logits)
    l2 = jnp.max(masked, axis=-1, keepdims=True)
    a2 = jnp.min(jnp.where(masked == l2, iota, e), axis=-1, keepdims=True)
    z = jnp.sum(jnp.exp(logits - l1), axis=-1, keepdims=True)
    p1 = 1.0 / z
    p2 = jnp.exp(l2 - l1) / z
    e2 = jnp.exp(p2 - p1)
    inv = 1.0 / (1.0 + e2)
    w_ref[...] = jnp.where(iota == a1, inv, 0.0) + jnp.where(iota == a2, e2 * inv, 0.0)


def _k2_body(q_ref, k_ref, v_ref, x_ref, wo_ref, o_ref, *, scale):
    qb = q_ref[0]
    scores = _dot_t(qb, k_ref[0]) * scale
    m = jnp.max(scores, axis=-1, keepdims=True)
    p = jnp.exp(scores - m)
    att = p / jnp.sum(p, axis=-1, keepdims=True)
    o = _dot(att, v_ref[0])
    o_ref[0] = x_ref[0] + _dot_t(o, wo_ref[...])


def _k3_body(*refs, n_experts):
    (y1_ref, h_ref, g_ref, w_ref) = refs[:4]
    eref = refs[4:4 + 4 * n_experts]
    (ws1_ref, bs1_ref, ws2_ref, bs2_ref, wu_ref, bu_ref, wc_ref, bc_ref,
     out_ref) = refs[4 + 4 * n_experts:]

    gb = g_ref[...]
    wb = w_ref[...]
    acc = jnp.zeros((gb.shape[0], gb.shape[1]), _F32)
    ctot = jnp.zeros((1, gb.shape[1]), _F32)
    for i in range(n_experts):
        w1 = eref[4 * i][...]
        b1 = eref[4 * i + 1][...]
        w2 = eref[4 * i + 2][...]
        b2 = eref[4 * i + 3][...]
        gb1 = _gelu(b1)
        t = _gelu(_dot_t(gb, w1) + b1) - gb1
        acc = acc + _dot_t(wb[:, i:i + 1] * t, w2)
        ctot = ctot + _dot_t(gb1, w2) + b2

    s = _gelu(_dot_t(gb, ws1_ref[...]) + bs1_ref[...])
    s = _dot_t(s, ws2_ref[...]) + bs2_ref[...]
    moe = acc + ctot + 0.1 * s

    up = _dot_t(moe, wu_ref[...]) + bu_ref[...]
    core = _dot_t(_gelu(h_ref[...]), wc_ref[...]) + bc_ref[...]
    out_ref[...] = y1_ref[...] + up + core


def kernel(x, rms_w, Wqkv, Wo, Wd, bd, Wu, bu, Wr, br, expert_params,
           Ws1, bs1, Ws2, bs2, Wc, bc):
    B, T, D = x.shape
    N = B * T
    L = Wd.shape[0]
    E = Wr.shape[0]
    hdims = [int(w1.shape[0]) for (w1, _, _, _) in expert_params]
    cost = jnp.asarray([2 * L * hd for hd in hdims], _F32)
    br_eff = (br - _COST_LAMBDA * cost).reshape(1, E)

    x2 = x.reshape(N, D)
    r2 = lambda a: a.reshape(1, -1)

    R1 = 512
    full = lambda arr: pl.BlockSpec(arr.shape, lambda i: (0,) * arr.ndim)
    row = lambda c: pl.BlockSpec((R1, c), lambda i: (i, 0))

    h, q, k, v, g, w = pl.pallas_call(
        _k1_body,
        grid=(N // R1,),
        in_specs=[row(D), full(r2(rms_w)), full(Wqkv), full(Wd),
                  full(r2(bd)), full(Wr), full(br_eff)],
        out_specs=[row(D), row(D), row(D), row(D), row(L), row(E)],
        out_shape=[
            jax.ShapeDtypeStruct((N, D), _F32),
            jax.ShapeDtypeStruct((N, D), _BF16),
            jax.ShapeDtypeStruct((N, D), _BF16),
            jax.ShapeDtypeStruct((N, D), _BF16),
            jax.ShapeDtypeStruct((N, L), _BF16),
            jax.ShapeDtypeStruct((N, E), _F32),
        ],
    )(x2, r2(rms_w), Wqkv, Wd, r2(bd), Wr, br_eff)

    RQ = 512
    q3 = q.reshape(B, T, D)
    k3 = k.reshape(B, T, D)
    v3 = v.reshape(B, T, D)
    qblk = pl.BlockSpec((1, RQ, D), lambda b, i: (b, i, 0))
    kvblk = pl.BlockSpec((1, T, D), lambda b, i: (b, 0, 0))
    y1 = pl.pallas_call(
        functools.partial(_k2_body, scale=D ** -0.5),
        grid=(B, T // RQ),
        in_specs=[qblk, kvblk, kvblk, qblk,
                  pl.BlockSpec(Wo.shape, lambda b, i: (0, 0))],
        out_specs=qblk,
        out_shape=jax.ShapeDtypeStruct((B, T, D), _F32),
    )(q3, k3, v3, x, Wo)

    R3 = 512
    row3 = lambda c: pl.BlockSpec((R3, c), lambda i: (i, 0))
    eops, especs = [], []
    for (w1, b1, w2, b2) in expert_params:
        for a in (w1, r2(b1), w2, r2(b2)):
            eops.append(a)
            especs.append(full(a))
    out = pl.pallas_call(
        functools.partial(_k3_body, n_experts=E),
        grid=(N // R3,),
        in_specs=[row3(D), row3(D), row3(L), row3(E)] + especs + [
            full(Ws1), full(r2(bs1)), full(Ws2), full(r2(bs2)),
            full(Wu), full(r2(bu)), full(Wc), full(r2(bc))],
        out_specs=row3(D),
        out_shape=jax.ShapeDtypeStruct((N, D), _F32),
    )(y1.reshape(N, D), h, g, w, *eops,
      Ws1, r2(bs1), Ws2, r2(bs2), Wu, r2(bu), Wc, r2(bc))

    return out.reshape(B, T, D)
